# baseline (speedup 1.0000x reference)
import functools

import jax
import jax.numpy as jnp
from jax import lax
from jax.experimental import pallas as pl
from jax.experimental.pallas import tpu as pltpu
from jax.experimental.pallas import tpu_sc as plsc

B, C, N = 16, 100, 1024
K = 9
CP = 128
EPS = 1e-5
IDX_ROWS = 16

NUM_WORKERS = 32
TOK_PER_WORKER = B * N // NUM_WORKERS
CHUNK = 64
CHUNKS_PER_WORKER = TOK_PER_WORKER // CHUNK


def _gelu_exact(x):
    return 0.5 * x * (1.0 + lax.erf(x * 0.7071067811865476))


def _row(v):
    return jnp.concatenate(
        [v.reshape(1, C), jnp.zeros((1, CP - C), jnp.float32)], axis=1)


def _phase_a_body(x_ref, w_ref, b_ref, g_ref, be_ref, ht_ref, idx_ref):
    xb = x_ref[0]
    w1p = jnp.concatenate(
        [w_ref[...], jnp.zeros((CP - C, C), jnp.float32)], axis=0)
    u = lax.dot_general(xb, w1p, (((0,), (1,)), ((), ())),
                        preferred_element_type=jnp.float32)
    u = u + _row(b_ref[...])
    ht = u / jnp.sqrt(jnp.float32(1.0 + EPS)) * _row(g_ref[...]) \
        + _row(be_ref[...])
    ht_ref[...] = ht

    inner = lax.dot_general(ht, ht, (((1,), (1,)), ((), ())),
                            preferred_element_type=jnp.float32)
    x2 = jnp.sum(ht * ht, axis=1, keepdims=True)
    iota = lax.broadcasted_iota(jnp.int32, (N, N), 0)
    eye = jnp.where(iota == lax.broadcasted_iota(jnp.int32, (N, N), 1),
                    jnp.float32(1.0), jnp.float32(0.0))
    x2_row = lax.dot_general(x2, eye, (((0,), (0,)), ((), ())),
                             preferred_element_type=jnp.float32,
                             precision=lax.Precision.HIGHEST)
    score = (x2_row - 2.0 * inner) + x2
    base = pl.program_id(0) * N
    for k in range(K):
        m = jnp.min(score, axis=0, keepdims=True)
        cand = jnp.where(score == m, iota, jnp.int32(2 ** 30))
        sel = jnp.min(cand, axis=0, keepdims=True)
        idx_ref[0, k:k + 1, :] = sel + base
        score = jnp.where(iota == sel, jnp.float32(jnp.inf), score)
    for k in range(K, IDX_ROWS):
        idx_ref[0, k:k + 1, :] = jnp.full((1, N), base, jnp.int32)


def _phase_a(x3, w1, b1, g1, be1):
    return pl.pallas_call(
        _phase_a_body,
        grid=(B,),
        in_specs=[
            pl.BlockSpec((1, C, N), lambda b: (b, 0, 0)),
            pl.BlockSpec((C, C), lambda b: (0, 0)),
            pl.BlockSpec((C,), lambda b: (0,)),
            pl.BlockSpec((C,), lambda b: (0,)),
            pl.BlockSpec((C,), lambda b: (0,)),
        ],
        out_specs=[
            pl.BlockSpec((N, CP), lambda b: (b, 0)),
            pl.BlockSpec((1, IDX_ROWS, N), lambda b: (b, 0, 0)),
        ],
        out_shape=[
            jax.ShapeDtypeStruct((B * N, CP), jnp.float32),
            jax.ShapeDtypeStruct((B, IDX_ROWS, N), jnp.int32),
        ],
    )(x3, w1, b1, g1, be1)


def _sc_gather_body(table_hbm, idx_hbm, out_hbm, idx_v, rows_v, out_v, sem):
    nc = 2
    wid = lax.axis_index("s") * nc + lax.axis_index("c")
    for chunk in range(CHUNKS_PER_WORKER):
        g0 = wid * TOK_PER_WORKER + chunk * CHUNK
        gc = g0 // CHUNK
        pltpu.sync_copy(
            idx_hbm.at[pl.ds(gc * (IDX_ROWS * CHUNK), K * CHUNK)], idx_v)
        copies = []
        for k in range(K):
            copies.append(
                pltpu.async_copy(
                    table_hbm.at[idx_v.at[pl.ds(k * CHUNK, CHUNK)]],
                    rows_v.at[k], sem))
        for cp in copies:
            cp.wait()

        def reduce_one(t, _):
            for g in range(CP // 16):
                sl = pl.ds(g * 16, 16)
                acc = rows_v[0, t, sl]
                for k in range(1, K):
                    acc = jnp.maximum(acc, rows_v[k, t, sl])
                out_v[t, sl] = acc
            return _

        lax.fori_loop(0, CHUNK, reduce_one, 0)
        pltpu.sync_copy(out_v, out_hbm.at[pl.ds(g0, CHUNK)])


def _phase_b(ht, idx):
    mesh = plsc.VectorSubcoreMesh(core_axis_name="c", subcore_axis_name="s")
    f = pl.kernel(
        _sc_gather_body,
        out_type=jax.ShapeDtypeStruct((B * N, CP), jnp.float32),
        mesh=mesh,
        scratch_types=[
            pltpu.VMEM((K * CHUNK,), jnp.int32),
            pltpu.VMEM((K, CHUNK, CP), jnp.float32),
            pltpu.VMEM((CHUNK, CP), jnp.float32),
            pltpu.SemaphoreType.DMA,
        ],
    )
    return f(ht, idx)


def _phase_c_body(x_ref, ht_ref, rel_ref, gcw_ref, sgc_ref, gcb_ref,
                  w2_ref, s2_ref, b2_ref, f1_ref, sf1_ref, bf1_ref,
                  f2_ref, sf2_ref, bf2_ref, out_ref):
    ht = ht_ref[...]
    relmh = rel_ref[...] - ht
    cat = jnp.concatenate([ht, relmh], axis=1)
    gs = gcw_ref[...] * sgc_ref[...]
    z = jnp.zeros((2 * C, CP - C), jnp.float32)
    gs2 = jnp.concatenate([gs[:, :C], z, gs[:, C:], z], axis=1)
    u = lax.dot_general(gs2, cat, (((1,), (1,)), ((), ())),
                        preferred_element_type=jnp.float32)
    u = _gelu_exact(u + gcb_ref[...])
    w2s = w2_ref[...] * s2_ref[...]
    y = lax.dot_general(w2s, u, (((1,), (0,)), ((), ())),
                        preferred_element_type=jnp.float32)
    h2 = y + b2_ref[...] + x_ref[0]
    f1s = f1_ref[...] * sf1_ref[...]
    t = _gelu_exact(
        lax.dot_general(f1s, h2, (((1,), (0,)), ((), ())),
                        preferred_element_type=jnp.float32) + bf1_ref[...])
    f2s = f2_ref[...] * sf2_ref[...]
    o = lax.dot_general(f2s, t, (((1,), (0,)), ((), ())),
                        preferred_element_type=jnp.float32) + bf2_ref[...] + h2
    out_ref[0] = o


def _phase_c(x3, ht, rel, gcw, sgc, gcb, w2, s2, b2, f1, sf1, bf1,
             f2, sf2, bf2):
    full = lambda shape: pl.BlockSpec(shape, lambda b: tuple(0 for _ in shape))
    return pl.pallas_call(
        _phase_c_body,
        grid=(B,),
        in_specs=[
            pl.BlockSpec((1, C, N), lambda b: (b, 0, 0)),
            pl.BlockSpec((N, CP), lambda b: (b, 0)),
            pl.BlockSpec((N, CP), lambda b: (b, 0)),
            full((2 * C, 2 * C)),
            full((2 * C, 1)),
            full((2 * C, 1)),
            full((C, 2 * C)),
            full((C, 1)),
            full((C, 1)),
            full((4 * C, C)),
            full((4 * C, 1)),
            full((4 * C, 1)),
            full((C, 4 * C)),
            full((C, 1)),
            full((C, 1)),
        ],
        out_specs=pl.BlockSpec((1, C, N), lambda b: (b, 0, 0)),
        out_shape=jax.ShapeDtypeStruct((B, C, N), jnp.float32),
    )(x3, ht, rel, gcw, sgc, gcb, w2, s2, b2, f1, sf1, bf1, f2, sf2, bf2)


@jax.jit
def kernel(x, g_fc1_w, g_fc1_b, g_bn1_g, g_bn1_b, gc_w, gc_b, gc_bn_g,
           gc_bn_b, g_fc2_w, g_fc2_b, g_bn2_g, g_bn2_b,
           f_fc1_w, f_fc1_b, f_bn1_g, f_bn1_b, f_fc2_w, f_fc2_b,
           f_bn2_g, f_bn2_b):
    x3 = x.reshape(B, C, N)
    scale = 1.0 / jnp.sqrt(jnp.float32(1.0 + EPS))

    def cols(bias, g, be):
        s = g * scale
        return s[:, None], (bias * s + be)[:, None]

    sgc, gcb = cols(gc_b, gc_bn_g, gc_bn_b)
    s2, b2 = cols(g_fc2_b, g_bn2_g, g_bn2_b)
    sf1, bf1 = cols(f_fc1_b, f_bn1_g, f_bn1_b)
    sf2, bf2 = cols(f_fc2_b, f_bn2_g, f_bn2_b)

    ht, idx = _phase_a(x3, g_fc1_w, g_fc1_b, g_bn1_g, g_bn1_b)
    idx_sc = idx.reshape(B, IDX_ROWS, N // CHUNK, CHUNK).transpose(
        0, 2, 1, 3).reshape(-1)
    rel = _phase_b(ht, idx_sc)
    out = _phase_c(x3, ht, rel, gc_w, sgc, gcb, g_fc2_w, s2, b2,
                   f_fc1_w, sf1, bf1, f_fc2_w, sf2, bf2)
    return out.reshape(x.shape)


# speedup vs baseline: 1.1103x; 1.1103x over previous
import functools

import jax
import jax.numpy as jnp
from jax import lax
from jax.experimental import pallas as pl
from jax.experimental.pallas import tpu as pltpu
from jax.experimental.pallas import tpu_sc as plsc

B, C, N = 16, 100, 1024
K = 9
CP = 128
EPS = 1e-5
IDX_ROWS = 16

NUM_WORKERS = 32
TOK_PER_WORKER = B * N // NUM_WORKERS
CHUNK = 64
CHUNKS_PER_WORKER = TOK_PER_WORKER // CHUNK


def _gelu_exact(x):
    return 0.5 * x * (1.0 + lax.erf(x * 0.7071067811865476))


def _row(v):
    return jnp.concatenate(
        [v.reshape(1, C), jnp.zeros((1, CP - C), jnp.float32)], axis=1)


def _phase_a_body(x_ref, w_ref, b_ref, g_ref, be_ref, ht_ref, idx_ref):
    xb = x_ref[0]
    w1p = jnp.concatenate(
        [w_ref[...], jnp.zeros((CP - C, C), jnp.float32)], axis=0)
    u = lax.dot_general(xb, w1p, (((0,), (1,)), ((), ())),
                        preferred_element_type=jnp.float32)
    u = u + _row(b_ref[...])
    ht = u / jnp.sqrt(jnp.float32(1.0 + EPS)) * _row(g_ref[...]) \
        + _row(be_ref[...])
    ht_ref[...] = ht

    inner = lax.dot_general(ht, ht, (((1,), (1,)), ((), ())),
                            preferred_element_type=jnp.float32)
    x2 = jnp.sum(ht * ht, axis=1, keepdims=True)
    iota = lax.broadcasted_iota(jnp.int32, (N, N), 0)
    eye = jnp.where(iota == lax.broadcasted_iota(jnp.int32, (N, N), 1),
                    jnp.float32(1.0), jnp.float32(0.0))
    x2_row = lax.dot_general(x2, eye, (((0,), (0,)), ((), ())),
                             preferred_element_type=jnp.float32,
                             precision=lax.Precision.HIGHEST)
    score = (x2_row - 2.0 * inner) + x2
    base = pl.program_id(0) * N
    for k in range(K):
        m = jnp.min(score, axis=0, keepdims=True)
        cand = jnp.where(score == m, iota, jnp.int32(2 ** 30))
        sel = jnp.min(cand, axis=0, keepdims=True)
        idx_ref[0, k:k + 1, :] = sel + base
        score = jnp.where(iota == sel, jnp.float32(jnp.inf), score)
    for k in range(K, IDX_ROWS):
        idx_ref[0, k:k + 1, :] = jnp.full((1, N), base, jnp.int32)


def _phase_a(x3, w1, b1, g1, be1, off, nb):
    return pl.pallas_call(
        _phase_a_body,
        grid=(nb,),
        in_specs=[
            pl.BlockSpec((1, C, N), lambda b: (b + off, 0, 0)),
            pl.BlockSpec((C, C), lambda b: (0, 0)),
            pl.BlockSpec((C,), lambda b: (0,)),
            pl.BlockSpec((C,), lambda b: (0,)),
            pl.BlockSpec((C,), lambda b: (0,)),
        ],
        out_specs=[
            pl.BlockSpec((N, CP), lambda b: (b, 0)),
            pl.BlockSpec((1, IDX_ROWS, N), lambda b: (b, 0, 0)),
        ],
        out_shape=[
            jax.ShapeDtypeStruct((nb * N, CP), jnp.float32),
            jax.ShapeDtypeStruct((nb, IDX_ROWS, N), jnp.int32),
        ],
    )(x3, w1, b1, g1, be1)


def _sc_gather_body(table_hbm, idx_hbm, out_hbm, idx_v, rows_v, out_v, sem,
                    *, tok_per_worker):
    nc = 2
    wid = lax.axis_index("s") * nc + lax.axis_index("c")
    for chunk in range(tok_per_worker // CHUNK):
        g0 = wid * tok_per_worker + chunk * CHUNK
        gc = g0 // CHUNK
        pltpu.sync_copy(
            idx_hbm.at[pl.ds(gc * (IDX_ROWS * CHUNK), K * CHUNK)], idx_v)
        copies = []
        for k in range(K):
            copies.append(
                pltpu.async_copy(
                    table_hbm.at[idx_v.at[pl.ds(k * CHUNK, CHUNK)]],
                    rows_v.at[k], sem))
        for cp in copies:
            cp.wait()

        def reduce_one(t, _):
            for g in range(CP // 16):
                sl = pl.ds(g * 16, 16)
                acc = rows_v[0, t, sl]
                for k in range(1, K):
                    acc = jnp.maximum(acc, rows_v[k, t, sl])
                out_v[t, sl] = acc
            return _

        lax.fori_loop(0, CHUNK, reduce_one, 0)
        pltpu.sync_copy(out_v, out_hbm.at[pl.ds(g0, CHUNK)])


def _phase_b(ht, idx):
    tokens = ht.shape[0]
    mesh = plsc.VectorSubcoreMesh(core_axis_name="c", subcore_axis_name="s")
    f = pl.kernel(
        functools.partial(_sc_gather_body,
                          tok_per_worker=tokens // NUM_WORKERS),
        out_type=jax.ShapeDtypeStruct((tokens, CP), jnp.float32),
        mesh=mesh,
        scratch_types=[
            pltpu.VMEM((K * CHUNK,), jnp.int32),
            pltpu.VMEM((K, CHUNK, CP), jnp.float32),
            pltpu.VMEM((CHUNK, CP), jnp.float32),
            pltpu.SemaphoreType.DMA,
        ],
    )
    return f(ht, idx)


def _phase_c_body(x_ref, ht_ref, rel_ref, gcw_ref, sgc_ref, gcb_ref,
                  w2_ref, s2_ref, b2_ref, f1_ref, sf1_ref, bf1_ref,
                  f2_ref, sf2_ref, bf2_ref, out_ref):
    ht = ht_ref[...]
    relmh = rel_ref[...] - ht
    cat = jnp.concatenate([ht, relmh], axis=1)
    gs = gcw_ref[...] * sgc_ref[...]
    z = jnp.zeros((2 * C, CP - C), jnp.float32)
    gs2 = jnp.concatenate([gs[:, :C], z, gs[:, C:], z], axis=1)
    u = lax.dot_general(gs2, cat, (((1,), (1,)), ((), ())),
                        preferred_element_type=jnp.float32)
    u = _gelu_exact(u + gcb_ref[...])
    w2s = w2_ref[...] * s2_ref[...]
    y = lax.dot_general(w2s, u, (((1,), (0,)), ((), ())),
                        preferred_element_type=jnp.float32)
    h2 = y + b2_ref[...] + x_ref[0]
    f1s = f1_ref[...] * sf1_ref[...]
    t = _gelu_exact(
        lax.dot_general(f1s, h2, (((1,), (0,)), ((), ())),
                        preferred_element_type=jnp.float32) + bf1_ref[...])
    f2s = f2_ref[...] * sf2_ref[...]
    o = lax.dot_general(f2s, t, (((1,), (0,)), ((), ())),
                        preferred_element_type=jnp.float32) + bf2_ref[...] + h2
    out_ref[0] = o


def _phase_c(x3, ht, rel, gcw, sgc, gcb, w2, s2, b2, f1, sf1, bf1,
             f2, sf2, bf2, off, nb):
    full = lambda shape: pl.BlockSpec(shape, lambda b: tuple(0 for _ in shape))
    return pl.pallas_call(
        _phase_c_body,
        grid=(nb,),
        in_specs=[
            pl.BlockSpec((1, C, N), lambda b: (b + off, 0, 0)),
            pl.BlockSpec((N, CP), lambda b: (b, 0)),
            pl.BlockSpec((N, CP), lambda b: (b, 0)),
            full((2 * C, 2 * C)),
            full((2 * C, 1)),
            full((2 * C, 1)),
            full((C, 2 * C)),
            full((C, 1)),
            full((C, 1)),
            full((4 * C, C)),
            full((4 * C, 1)),
            full((4 * C, 1)),
            full((C, 4 * C)),
            full((C, 1)),
            full((C, 1)),
        ],
        out_specs=pl.BlockSpec((1, C, N), lambda b: (b, 0, 0)),
        out_shape=jax.ShapeDtypeStruct((nb, C, N), jnp.float32),
    )(x3, ht, rel, gcw, sgc, gcb, w2, s2, b2, f1, sf1, bf1, f2, sf2, bf2)


@jax.jit
def kernel(x, g_fc1_w, g_fc1_b, g_bn1_g, g_bn1_b, gc_w, gc_b, gc_bn_g,
           gc_bn_b, g_fc2_w, g_fc2_b, g_bn2_g, g_bn2_b,
           f_fc1_w, f_fc1_b, f_bn1_g, f_bn1_b, f_fc2_w, f_fc2_b,
           f_bn2_g, f_bn2_b):
    x3 = x.reshape(B, C, N)
    scale = 1.0 / jnp.sqrt(jnp.float32(1.0 + EPS))

    def cols(bias, g, be):
        s = g * scale
        return s[:, None], (bias * s + be)[:, None]

    sgc, gcb = cols(gc_b, gc_bn_g, gc_bn_b)
    s2, b2 = cols(g_fc2_b, g_bn2_g, g_bn2_b)
    sf1, bf1 = cols(f_fc1_b, f_bn1_g, f_bn1_b)
    sf2, bf2 = cols(f_fc2_b, f_bn2_g, f_bn2_b)

    nb = B // 2
    hts, idxs, rels, outs = [], [], [], []
    for h in range(2):
        ht, idx = _phase_a(x3, g_fc1_w, g_fc1_b, g_bn1_g, g_bn1_b,
                           h * nb, nb)
        idx_sc = idx.reshape(nb, IDX_ROWS, N // CHUNK, CHUNK).transpose(
            0, 2, 1, 3).reshape(-1)
        hts.append(ht)
        rels.append(_phase_b(ht, idx_sc))
    for h in range(2):
        outs.append(_phase_c(x3, hts[h], rels[h], gc_w, sgc, gcb,
                             g_fc2_w, s2, b2, f_fc1_w, sf1, bf1,
                             f_fc2_w, sf2, bf2, h * nb, nb))
    out = jnp.concatenate(outs, axis=0)
    return out.reshape(x.shape)


# speedup vs baseline: 1.1212x; 1.0098x over previous
import functools

import jax
import jax.numpy as jnp
from jax import lax
from jax.experimental import pallas as pl
from jax.experimental.pallas import tpu as pltpu
from jax.experimental.pallas import tpu_sc as plsc

B, C, N = 16, 100, 1024
K = 9
CP = 128
EPS = 1e-5
IDX_ROWS = 16

NUM_WORKERS = 32
TOK_PER_WORKER = B * N // NUM_WORKERS
CHUNK = 64
CHUNKS_PER_WORKER = TOK_PER_WORKER // CHUNK


def _gelu_exact(x):
    return 0.5 * x * (1.0 + lax.erf(x * 0.7071067811865476))


def _row(v):
    return jnp.concatenate(
        [v.reshape(1, C), jnp.zeros((1, CP - C), jnp.float32)], axis=1)


def _phase_a_body(x_ref, w_ref, b_ref, g_ref, be_ref, ht_ref, idx_ref):
    xb = x_ref[0]
    w1p = jnp.concatenate(
        [w_ref[...], jnp.zeros((CP - C, C), jnp.float32)], axis=0)
    u = lax.dot_general(xb, w1p, (((0,), (1,)), ((), ())),
                        preferred_element_type=jnp.float32)
    u = u + _row(b_ref[...])
    ht = u / jnp.sqrt(jnp.float32(1.0 + EPS)) * _row(g_ref[...]) \
        + _row(be_ref[...])
    ht_ref[...] = ht

    inner = lax.dot_general(ht, ht, (((1,), (1,)), ((), ())),
                            preferred_element_type=jnp.float32)
    x2 = jnp.sum(ht * ht, axis=1, keepdims=True)
    iota = lax.broadcasted_iota(jnp.int32, (N, N), 0)
    eye = jnp.where(iota == lax.broadcasted_iota(jnp.int32, (N, N), 1),
                    jnp.float32(1.0), jnp.float32(0.0))
    x2_row = lax.dot_general(x2, eye, (((0,), (0,)), ((), ())),
                             preferred_element_type=jnp.float32,
                             precision=lax.Precision.HIGHEST)
    score = (x2_row - 2.0 * inner) + x2
    base = pl.program_id(0) * N
    for k in range(K):
        m = jnp.min(score, axis=0, keepdims=True)
        cand = jnp.where(score == m, iota, jnp.int32(2 ** 30))
        sel = jnp.min(cand, axis=0, keepdims=True)
        idx_ref[0, k:k + 1, :] = sel + base
        score = jnp.where(iota == sel, jnp.float32(jnp.inf), score)
    for k in range(K, IDX_ROWS):
        idx_ref[0, k:k + 1, :] = jnp.full((1, N), base, jnp.int32)


def _phase_a(x3, w1, b1, g1, be1, off, nb):
    return pl.pallas_call(
        _phase_a_body,
        grid=(nb,),
        in_specs=[
            pl.BlockSpec((1, C, N), lambda b: (b + off, 0, 0)),
            pl.BlockSpec((C, C), lambda b: (0, 0)),
            pl.BlockSpec((C,), lambda b: (0,)),
            pl.BlockSpec((C,), lambda b: (0,)),
            pl.BlockSpec((C,), lambda b: (0,)),
        ],
        out_specs=[
            pl.BlockSpec((N, CP), lambda b: (b, 0)),
            pl.BlockSpec((1, IDX_ROWS, N), lambda b: (b, 0, 0)),
        ],
        out_shape=[
            jax.ShapeDtypeStruct((nb * N, CP), jnp.float32),
            jax.ShapeDtypeStruct((nb, IDX_ROWS, N), jnp.int32),
        ],
    )(x3, w1, b1, g1, be1)


def _sc_gather_body(table_hbm, idx_hbm, out_hbm, idx_v, rows_v, out_v, sem,
                    *, tok_per_worker):
    nc = 2
    wid = lax.axis_index("s") * nc + lax.axis_index("c")
    for chunk in range(tok_per_worker // CHUNK):
        g0 = wid * tok_per_worker + chunk * CHUNK
        gc = g0 // CHUNK
        pltpu.sync_copy(
            idx_hbm.at[pl.ds(gc * (IDX_ROWS * CHUNK), K * CHUNK)], idx_v)
        copies = []
        for k in range(K):
            copies.append(
                pltpu.async_copy(
                    table_hbm.at[idx_v.at[pl.ds(k * CHUNK, CHUNK)]],
                    rows_v.at[k], sem))
        for cp in copies:
            cp.wait()

        def reduce_one(t, _):
            for g in range(CP // 16):
                sl = pl.ds(g * 16, 16)
                acc = rows_v[0, t, sl]
                for k in range(1, K):
                    acc = jnp.maximum(acc, rows_v[k, t, sl])
                out_v[t, sl] = acc
            return _

        lax.fori_loop(0, CHUNK, reduce_one, 0)
        pltpu.sync_copy(out_v, out_hbm.at[pl.ds(g0, CHUNK)])


def _phase_b(ht, idx):
    tokens = ht.shape[0]
    mesh = plsc.VectorSubcoreMesh(core_axis_name="c", subcore_axis_name="s")
    f = pl.kernel(
        functools.partial(_sc_gather_body,
                          tok_per_worker=tokens // NUM_WORKERS),
        out_type=jax.ShapeDtypeStruct((tokens, CP), jnp.float32),
        mesh=mesh,
        scratch_types=[
            pltpu.VMEM((K * CHUNK,), jnp.int32),
            pltpu.VMEM((K, CHUNK, CP), jnp.float32),
            pltpu.VMEM((CHUNK, CP), jnp.float32),
            pltpu.SemaphoreType.DMA,
        ],
    )
    return f(ht, idx)


def _phase_c_body(x_ref, ht_ref, rel_ref, gcw_ref, sgc_ref, gcb_ref,
                  w2_ref, s2_ref, b2_ref, f1_ref, sf1_ref, bf1_ref,
                  f2_ref, sf2_ref, bf2_ref, out_ref):
    ht = ht_ref[...]
    relmh = rel_ref[...] - ht
    cat = jnp.concatenate([ht, relmh], axis=1)
    gs = gcw_ref[...] * sgc_ref[...]
    z = jnp.zeros((2 * C, CP - C), jnp.float32)
    gs2 = jnp.concatenate([gs[:, :C], z, gs[:, C:], z], axis=1)
    u = lax.dot_general(gs2, cat, (((1,), (1,)), ((), ())),
                        preferred_element_type=jnp.float32)
    u = _gelu_exact(u + gcb_ref[...])
    w2s = w2_ref[...] * s2_ref[...]
    y = lax.dot_general(w2s, u, (((1,), (0,)), ((), ())),
                        preferred_element_type=jnp.float32)
    h2 = y + b2_ref[...] + x_ref[0]
    f1s = f1_ref[...] * sf1_ref[...]
    t = _gelu_exact(
        lax.dot_general(f1s, h2, (((1,), (0,)), ((), ())),
                        preferred_element_type=jnp.float32) + bf1_ref[...])
    f2s = f2_ref[...] * sf2_ref[...]
    o = lax.dot_general(f2s, t, (((1,), (0,)), ((), ())),
                        preferred_element_type=jnp.float32) + bf2_ref[...] + h2
    out_ref[0] = o


def _phase_c(x3, ht, rel, gcw, sgc, gcb, w2, s2, b2, f1, sf1, bf1,
             f2, sf2, bf2, off, nb):
    full = lambda shape: pl.BlockSpec(shape, lambda b: tuple(0 for _ in shape))
    return pl.pallas_call(
        _phase_c_body,
        grid=(nb,),
        in_specs=[
            pl.BlockSpec((1, C, N), lambda b: (b + off, 0, 0)),
            pl.BlockSpec((N, CP), lambda b: (b, 0)),
            pl.BlockSpec((N, CP), lambda b: (b, 0)),
            full((2 * C, 2 * C)),
            full((2 * C, 1)),
            full((2 * C, 1)),
            full((C, 2 * C)),
            full((C, 1)),
            full((C, 1)),
            full((4 * C, C)),
            full((4 * C, 1)),
            full((4 * C, 1)),
            full((C, 4 * C)),
            full((C, 1)),
            full((C, 1)),
        ],
        out_specs=pl.BlockSpec((1, C, N), lambda b: (b, 0, 0)),
        out_shape=jax.ShapeDtypeStruct((nb, C, N), jnp.float32),
    )(x3, ht, rel, gcw, sgc, gcb, w2, s2, b2, f1, sf1, bf1, f2, sf2, bf2)


@jax.jit
def kernel(x, g_fc1_w, g_fc1_b, g_bn1_g, g_bn1_b, gc_w, gc_b, gc_bn_g,
           gc_bn_b, g_fc2_w, g_fc2_b, g_bn2_g, g_bn2_b,
           f_fc1_w, f_fc1_b, f_bn1_g, f_bn1_b, f_fc2_w, f_fc2_b,
           f_bn2_g, f_bn2_b):
    x3 = x.reshape(B, C, N)
    scale = 1.0 / jnp.sqrt(jnp.float32(1.0 + EPS))

    def cols(bias, g, be):
        s = g * scale
        return s[:, None], (bias * s + be)[:, None]

    sgc, gcb = cols(gc_b, gc_bn_g, gc_bn_b)
    s2, b2 = cols(g_fc2_b, g_bn2_g, g_bn2_b)
    sf1, bf1 = cols(f_fc1_b, f_bn1_g, f_bn1_b)
    sf2, bf2 = cols(f_fc2_b, f_bn2_g, f_bn2_b)

    nsplit = 4
    nb = B // nsplit
    hts, idxs, rels, outs = [], [], [], []
    for h in range(nsplit):
        ht, idx = _phase_a(x3, g_fc1_w, g_fc1_b, g_bn1_g, g_bn1_b,
                           h * nb, nb)
        idx_sc = idx.reshape(nb, IDX_ROWS, N // CHUNK, CHUNK).transpose(
            0, 2, 1, 3).reshape(-1)
        hts.append(ht)
        rels.append(_phase_b(ht, idx_sc))
    for h in range(nsplit):
        outs.append(_phase_c(x3, hts[h], rels[h], gc_w, sgc, gcb,
                             g_fc2_w, s2, b2, f_fc1_w, sf1, bf1,
                             f_fc2_w, sf2, bf2, h * nb, nb))
    out = jnp.concatenate(outs, axis=0)
    return out.reshape(x.shape)


# speedup vs baseline: 1.1282x; 1.0062x over previous
import functools

import jax
import jax.numpy as jnp
from jax import lax
from jax.experimental import pallas as pl
from jax.experimental.pallas import tpu as pltpu
from jax.experimental.pallas import tpu_sc as plsc

B, C, N = 16, 100, 1024
K = 9
CP = 128
EPS = 1e-5
IDX_ROWS = 16

NUM_WORKERS = 32
TOK_PER_WORKER = B * N // NUM_WORKERS
CHUNK = 64
CHUNKS_PER_WORKER = TOK_PER_WORKER // CHUNK


def _gelu_exact(x):
    return 0.5 * x * (1.0 + lax.erf(x * 0.7071067811865476))


def _row(v):
    return jnp.concatenate(
        [v.reshape(1, C), jnp.zeros((1, CP - C), jnp.float32)], axis=1)


def _phase_a_body(x_ref, w_ref, b_ref, g_ref, be_ref, ht_ref, idx_ref):
    xb = x_ref[0]
    w1p = jnp.concatenate(
        [w_ref[...], jnp.zeros((CP - C, C), jnp.float32)], axis=0)
    u = lax.dot_general(xb, w1p, (((0,), (1,)), ((), ())),
                        preferred_element_type=jnp.float32)
    u = u + _row(b_ref[...])
    ht = u / jnp.sqrt(jnp.float32(1.0 + EPS)) * _row(g_ref[...]) \
        + _row(be_ref[...])
    ht_ref[...] = ht

    inner = lax.dot_general(ht, ht, (((1,), (1,)), ((), ())),
                            preferred_element_type=jnp.float32)
    x2 = jnp.sum(ht * ht, axis=1, keepdims=True)
    iota = lax.broadcasted_iota(jnp.int32, (N, N), 0)
    eye = jnp.where(iota == lax.broadcasted_iota(jnp.int32, (N, N), 1),
                    jnp.float32(1.0), jnp.float32(0.0))
    x2_row = lax.dot_general(x2, eye, (((0,), (0,)), ((), ())),
                             preferred_element_type=jnp.float32,
                             precision=lax.Precision.HIGHEST)
    score = (x2_row - 2.0 * inner) + x2
    base = pl.program_id(0) * N
    for k in range(K):
        m = jnp.min(score, axis=0, keepdims=True)
        cand = jnp.where(score == m, iota, jnp.int32(2 ** 30))
        sel = jnp.min(cand, axis=0, keepdims=True)
        idx_ref[0, k:k + 1, :] = sel + base
        score = jnp.where(iota == sel, jnp.float32(jnp.inf), score)
    for k in range(K, IDX_ROWS):
        idx_ref[0, k:k + 1, :] = jnp.full((1, N), base, jnp.int32)


def _phase_a(x3, w1, b1, g1, be1, off, nb):
    return pl.pallas_call(
        _phase_a_body,
        grid=(nb,),
        in_specs=[
            pl.BlockSpec((1, C, N), lambda b: (b + off, 0, 0)),
            pl.BlockSpec((C, C), lambda b: (0, 0)),
            pl.BlockSpec((C,), lambda b: (0,)),
            pl.BlockSpec((C,), lambda b: (0,)),
            pl.BlockSpec((C,), lambda b: (0,)),
        ],
        out_specs=[
            pl.BlockSpec((N, CP), lambda b: (b, 0)),
            pl.BlockSpec((1, IDX_ROWS, N), lambda b: (b, 0, 0)),
        ],
        out_shape=[
            jax.ShapeDtypeStruct((nb * N, CP), jnp.float32),
            jax.ShapeDtypeStruct((nb, IDX_ROWS, N), jnp.int32),
        ],
    )(x3, w1, b1, g1, be1)


def _sc_gather_body(table_hbm, idx_hbm, out_hbm, idx_v, rows_v, out_v, sem,
                    *, tok_per_worker):
    nc = 2
    wid = lax.axis_index("s") * nc + lax.axis_index("c")
    for chunk in range(tok_per_worker // CHUNK):
        g0 = wid * tok_per_worker + chunk * CHUNK
        b = g0 // N
        n0 = g0 % N
        icopies = [
            pltpu.async_copy(
                idx_hbm.at[pl.ds(b * (IDX_ROWS * N) + k * N + n0, CHUNK)],
                idx_v.at[k], sem)
            for k in range(K)]
        for cp in icopies:
            cp.wait()
        copies = []
        for k in range(K):
            copies.append(
                pltpu.async_copy(table_hbm.at[idx_v.at[k]],
                                 rows_v.at[k], sem))
        for cp in copies:
            cp.wait()

        def reduce_one(t, _):
            for g in range(CP // 16):
                sl = pl.ds(g * 16, 16)
                acc = rows_v[0, t, sl]
                for k in range(1, K):
                    acc = jnp.maximum(acc, rows_v[k, t, sl])
                out_v[t, sl] = acc
            return _

        lax.fori_loop(0, CHUNK, reduce_one, 0)
        pltpu.sync_copy(out_v, out_hbm.at[pl.ds(g0, CHUNK)])


def _phase_b(ht, idx):
    tokens = ht.shape[0]
    mesh = plsc.VectorSubcoreMesh(core_axis_name="c", subcore_axis_name="s")
    f = pl.kernel(
        functools.partial(_sc_gather_body,
                          tok_per_worker=tokens // NUM_WORKERS),
        out_type=jax.ShapeDtypeStruct((tokens, CP), jnp.float32),
        mesh=mesh,
        scratch_types=[
            pltpu.VMEM((K, CHUNK), jnp.int32),
            pltpu.VMEM((K, CHUNK, CP), jnp.float32),
            pltpu.VMEM((CHUNK, CP), jnp.float32),
            pltpu.SemaphoreType.DMA,
        ],
    )
    return f(ht, idx)


def _phase_c_body(x_ref, ht_ref, rel_ref, gcw_ref, sgc_ref, gcb_ref,
                  w2_ref, s2_ref, b2_ref, f1_ref, sf1_ref, bf1_ref,
                  f2_ref, sf2_ref, bf2_ref, out_ref):
    ht = ht_ref[...]
    relmh = rel_ref[...] - ht
    cat = jnp.concatenate([ht, relmh], axis=1)
    gs = gcw_ref[...] * sgc_ref[...]
    z = jnp.zeros((2 * C, CP - C), jnp.float32)
    gs2 = jnp.concatenate([gs[:, :C], z, gs[:, C:], z], axis=1)
    u = lax.dot_general(gs2, cat, (((1,), (1,)), ((), ())),
                        preferred_element_type=jnp.float32)
    u = _gelu_exact(u + gcb_ref[...])
    w2s = w2_ref[...] * s2_ref[...]
    y = lax.dot_general(w2s, u, (((1,), (0,)), ((), ())),
                        preferred_element_type=jnp.float32)
    h2 = y + b2_ref[...] + x_ref[0]
    f1s = f1_ref[...] * sf1_ref[...]
    t = _gelu_exact(
        lax.dot_general(f1s, h2, (((1,), (0,)), ((), ())),
                        preferred_element_type=jnp.float32) + bf1_ref[...])
    f2s = f2_ref[...] * sf2_ref[...]
    o = lax.dot_general(f2s, t, (((1,), (0,)), ((), ())),
                        preferred_element_type=jnp.float32) + bf2_ref[...] + h2
    out_ref[0] = o


def _phase_c(x3, ht, rel, gcw, sgc, gcb, w2, s2, b2, f1, sf1, bf1,
             f2, sf2, bf2, off, nb):
    full = lambda shape: pl.BlockSpec(shape, lambda b: tuple(0 for _ in shape))
    return pl.pallas_call(
        _phase_c_body,
        grid=(nb,),
        in_specs=[
            pl.BlockSpec((1, C, N), lambda b: (b + off, 0, 0)),
            pl.BlockSpec((N, CP), lambda b: (b, 0)),
            pl.BlockSpec((N, CP), lambda b: (b, 0)),
            full((2 * C, 2 * C)),
            full((2 * C, 1)),
            full((2 * C, 1)),
            full((C, 2 * C)),
            full((C, 1)),
            full((C, 1)),
            full((4 * C, C)),
            full((4 * C, 1)),
            full((4 * C, 1)),
            full((C, 4 * C)),
            full((C, 1)),
            full((C, 1)),
        ],
        out_specs=pl.BlockSpec((1, C, N), lambda b: (b, 0, 0)),
        out_shape=jax.ShapeDtypeStruct((nb, C, N), jnp.float32),
    )(x3, ht, rel, gcw, sgc, gcb, w2, s2, b2, f1, sf1, bf1, f2, sf2, bf2)


@jax.jit
def kernel(x, g_fc1_w, g_fc1_b, g_bn1_g, g_bn1_b, gc_w, gc_b, gc_bn_g,
           gc_bn_b, g_fc2_w, g_fc2_b, g_bn2_g, g_bn2_b,
           f_fc1_w, f_fc1_b, f_bn1_g, f_bn1_b, f_fc2_w, f_fc2_b,
           f_bn2_g, f_bn2_b):
    x3 = x.reshape(B, C, N)
    scale = 1.0 / jnp.sqrt(jnp.float32(1.0 + EPS))

    def cols(bias, g, be):
        s = g * scale
        return s[:, None], (bias * s + be)[:, None]

    sgc, gcb = cols(gc_b, gc_bn_g, gc_bn_b)
    s2, b2 = cols(g_fc2_b, g_bn2_g, g_bn2_b)
    sf1, bf1 = cols(f_fc1_b, f_bn1_g, f_bn1_b)
    sf2, bf2 = cols(f_fc2_b, f_bn2_g, f_bn2_b)

    nsplit = 4
    nb = B // nsplit
    hts, idxs, rels, outs = [], [], [], []
    for h in range(nsplit):
        ht, idx = _phase_a(x3, g_fc1_w, g_fc1_b, g_bn1_g, g_bn1_b,
                           h * nb, nb)
        hts.append(ht)
        rels.append(_phase_b(ht, idx.reshape(-1)))
    for h in range(nsplit):
        outs.append(_phase_c(x3, hts[h], rels[h], gc_w, sgc, gcb,
                             g_fc2_w, s2, b2, f_fc1_w, sf1, bf1,
                             f_fc2_w, sf2, bf2, h * nb, nb))
    out = jnp.concatenate(outs, axis=0)
    return out.reshape(x.shape)
